# baseline (speedup 1.0000x reference)
import functools
import math

import jax
import jax.numpy as jnp
from jax import lax
from jax.experimental import pallas as pl
from jax.experimental.pallas import tpu as pltpu
from jax.experimental.pallas import tpu_sc as plsc

D = 512

_NW = 32
_R_SC = 512
_RPW = _R_SC // _NW
_LTOK = 64
_DH = 768
_NCH = _DH // 16


def _pool_sc_body(s_hbm, out_hbm, buf_v, outv_v, sem):
    wid = lax.axis_index("s") * 2 + lax.axis_index("c")
    base = wid * _RPW

    pltpu.async_copy(s_hbm.at[base], buf_v.at[pl.ds(0, _LTOK)], sem)

    def sent_body(i, carry):
        p = lax.rem(i, 2) * _LTOK
        row = base + i
        pltpu.make_async_copy(
            s_hbm.at[row], buf_v.at[pl.ds(p, _LTOK)], sem).wait()

        @pl.when(i + 1 < _RPW)
        def _():
            pn = lax.rem(i + 1, 2) * _LTOK
            pltpu.async_copy(s_hbm.at[row + 1],
                             buf_v.at[pl.ds(pn, _LTOK)], sem)

        G = 12
        for g in range(_NCH // G):
            def tok_body(l2, accs, g=g):
                l = p + 2 * l2
                a = tuple(
                    accs[j] + buf_v[l, pl.ds(16 * (G * g + j), 16)]
                    for j in range(G))
                return tuple(
                    a[j] + buf_v[l + 1, pl.ds(16 * (G * g + j), 16)]
                    for j in range(G))
            z = jnp.zeros((16,), jnp.float32)
            accs = lax.fori_loop(0, _LTOK // 2, tok_body, (z,) * G)
            for j in range(G):
                outv_v[i, pl.ds(16 * (G * g + j), 16)] = accs[j]
        return carry

    lax.fori_loop(0, _RPW, sent_body, 0)
    pltpu.sync_copy(outv_v, out_hbm.at[pl.ds(base, _RPW)])


def _pool_sc(sentences_hidden3):
    mesh = plsc.VectorSubcoreMesh(core_axis_name="c", subcore_axis_name="s")
    kfn = pl.kernel(
        _pool_sc_body,
        mesh=mesh,
        out_type=jax.ShapeDtypeStruct((_R_SC, _DH), jnp.float32),
        scratch_types=[
            pltpu.VMEM((2 * _LTOK, _DH), jnp.float32),
            pltpu.VMEM((_RPW, _DH), jnp.float32),
            pltpu.SemaphoreType.DMA,
        ],
    )
    return kfn(sentences_hidden3)


def _pool_body(s_ref, out_ref):
    out_ref[...] = jnp.sum(s_ref[...], axis=1)


def _pool_tc_tail(sentences_hidden3, rows_per_block=16):
    BS, L, DH = sentences_hidden3.shape
    ntail = BS - _R_SC
    nblk = ntail // rows_per_block
    off = _R_SC // rows_per_block
    return pl.pallas_call(
        _pool_body,
        grid=(nblk,),
        in_specs=[
            pl.BlockSpec((rows_per_block, L, DH), lambda i: (i + off, 0, 0)),
        ],
        out_specs=pl.BlockSpec((rows_per_block, DH), lambda i: (i, 0)),
        out_shape=jax.ShapeDtypeStruct((ntail, DH), jnp.float32),
    )(sentences_hidden3)


def _fused_body(ps_sc_ref, ps_tc_ref, mask_ref, ht_ref, nq_ref, adj_ref,
                et_ref, W_hp_ref, b_hp_ref, W_ql_ref, b_ql_ref, W_kl_ref,
                b_kl_ref, g_q_ref, beta_q_ref, g_k_ref, beta_k_ref,
                flag_ref, edge_ref, Wq_ref, Wk_ref, Wv_ref, We_ref,
                hidden_ref, recall_ref, *, B, S):
    bf = jnp.bfloat16
    ps = jnp.concatenate([ps_sc_ref[...], ps_tc_ref[...]], axis=0)
    msum = jnp.sum(mask_ref[...], axis=1, keepdims=True)
    inv = 1.0 / jnp.where(msum != 0.0, msum, 1.0)
    node = jnp.dot(ps.astype(bf), W_hp_ref[...],
                   preferred_element_type=jnp.float32) * inv + b_hp_ref[...]
    ht = ht_ref[...].astype(jnp.float32)
    f0 = flag_ref[0:1, :]
    f1 = flag_ref[1:2, :]
    h = node + f0 + ht * (f1 - f0)
    hb = h.astype(bf)
    q = jnp.dot(hb, Wq_ref[...], preferred_element_type=jnp.float32)
    k = jnp.dot(hb, Wk_ref[...], preferred_element_type=jnp.float32)
    v = jnp.dot(hb, Wv_ref[...], preferred_element_type=jnp.float32)
    ew = jnp.dot(edge_ref[...], We_ref[...],
                 preferred_element_type=jnp.float32)
    ql = jnp.dot(nq_ref[...], W_ql_ref[...],
                 preferred_element_type=jnp.float32) + b_ql_ref[...]
    mu = jnp.mean(ql, axis=-1, keepdims=True)
    var = jnp.mean((ql - mu) ** 2, axis=-1, keepdims=True)
    query = ((ql - mu) / jnp.sqrt(var + 1e-5)) * g_q_ref[...] \
        + beta_q_ref[...]

    dn = (((1,), (1,)), ((), ()))
    qb = q.astype(bf)
    kb = k.astype(bf)
    vb = v.astype(bf)
    qe = lax.dot_general(qb, ew.astype(bf), dn,
                         preferred_element_type=jnp.float32)
    isq = 1.0 / math.sqrt(float(D))
    neg = jnp.float32(-1e9)
    outs = []
    for j in range(B):
        sl = slice(j * S, (j + 1) * S)
        adj = adj_ref[j]
        et = et_ref[j]
        scores = lax.dot_general(qb[sl], kb[sl], dn,
                                 preferred_element_type=jnp.float32)
        esc = jnp.zeros_like(scores)
        qej = qe[sl]
        for t in range(5):
            esc = jnp.where(et == t,
                            jnp.broadcast_to(qej[:, t:t + 1], scores.shape),
                            esc)
        scores = (scores + esc) * isq
        scores = jnp.where(adj > 0, scores, neg)
        mx = jnp.max(scores, axis=-1, keepdims=True)
        p = jnp.exp(scores - mx)
        attn = p / jnp.sum(p, axis=-1, keepdims=True)
        row_has = (jnp.sum(adj.astype(jnp.float32), axis=-1, keepdims=True)
                   > 0.0).astype(jnp.float32)
        attn = attn * row_has
        outs.append(jnp.dot(attn.astype(bf), vb[sl],
                            preferred_element_type=jnp.float32))
    hidden = jnp.concatenate(outs, axis=0) + h
    for j in range(B):
        hidden_ref[j] = hidden[j * S:(j + 1) * S]
    kl = jnp.dot(hidden.astype(bf), W_kl_ref[...],
                 preferred_element_type=jnp.float32) + b_kl_ref[...]
    mu = jnp.mean(kl, axis=-1, keepdims=True)
    var = jnp.mean((kl - mu) ** 2, axis=-1, keepdims=True)
    key = ((kl - mu) / jnp.sqrt(var + 1e-5)) * g_k_ref[...] + beta_k_ref[...]
    pad = (jnp.sum(mask_ref[...], axis=-1) != 0.0).astype(jnp.float32)
    for j in range(B):
        sl = slice(j * S, (j + 1) * S)
        logits = jnp.sum(key[sl] * query[j:j + 1, :], axis=-1)
        recall_ref[j:j + 1, :] = (jax.nn.sigmoid(logits) * pad[sl])[None, :]


def _fused(ps_sc, ps_tc, mask, head_flat, node_query, adj, et,
           W_hp, b_hp, W_ql, b_ql, W_kl, b_kl, g_q, beta_q, g_k, beta_k,
           flag_embed, edge_embed, Wq, Wk, Wv, We):
    B = adj.shape[0]
    S = adj.shape[1]
    outs = (
        jax.ShapeDtypeStruct((B, S, D), jnp.float32),
        jax.ShapeDtypeStruct((B, S), jnp.float32),
    )
    return pl.pallas_call(
        functools.partial(_fused_body, B=B, S=S), out_shape=outs)(
        ps_sc, ps_tc, mask, head_flat, node_query, adj, et, W_hp, b_hp,
        W_ql, b_ql, W_kl, b_kl, g_q, beta_q, g_k, beta_k, flag_embed,
        edge_embed, Wq, Wk, Wv, We)


def kernel(sentences_hidden, sentences_num, sentences_mask,
           sent_adjacent_matrix, head_type, edge_type, node_query,
           W_hp, b_hp, W_ql, b_ql, W_kl, b_kl, g_q, beta_q, g_k, beta_k,
           flag_embed, edge_embed, Wq, Wk, Wv, We):
    BS, L, DH = sentences_hidden.shape
    B = sentences_num.shape[0]
    S = BS // B

    s3 = sentences_hidden.reshape(BS, L, DH)
    ps_sc = _pool_sc(s3)
    ps_tc = _pool_tc_tail(s3)

    head_flat = head_type.reshape(BS, 1).astype(jnp.int32)
    r1 = lambda x: x.reshape(1, -1)
    adj = sent_adjacent_matrix.astype(jnp.int32)
    et = edge_type.astype(jnp.int32)
    bf = jnp.bfloat16
    hidden, recall = _fused(
        ps_sc, ps_tc, sentences_mask, head_flat, node_query, adj, et,
        W_hp.astype(bf), r1(b_hp), W_ql, r1(b_ql), W_kl.astype(bf),
        r1(b_kl), r1(g_q), r1(beta_q), r1(g_k), r1(beta_k), flag_embed,
        edge_embed, Wq.astype(bf), Wk.astype(bf), Wv.astype(bf), We)
    return recall, hidden


# speedup vs baseline: 1.0055x; 1.0055x over previous
import functools
import math

import jax
import jax.numpy as jnp
from jax import lax
from jax.experimental import pallas as pl
from jax.experimental.pallas import tpu as pltpu
from jax.experimental.pallas import tpu_sc as plsc

D = 512

_NW = 32
_R_SC = 512
_RPW = _R_SC // _NW
_LTOK = 64
_DH = 768
_NCH = _DH // 16


def _pool_sc_body(s_hbm, out_hbm, buf_v, outv_v, sem):
    wid = lax.axis_index("s") * 2 + lax.axis_index("c")
    base = wid * _RPW

    pltpu.async_copy(s_hbm.at[base], buf_v.at[pl.ds(0, _LTOK)], sem)

    def sent_body(i, carry):
        p = lax.rem(i, 2) * _LTOK
        row = base + i
        pltpu.make_async_copy(
            s_hbm.at[row], buf_v.at[pl.ds(p, _LTOK)], sem).wait()

        @pl.when(i + 1 < _RPW)
        def _():
            pn = lax.rem(i + 1, 2) * _LTOK
            pltpu.async_copy(s_hbm.at[row + 1],
                             buf_v.at[pl.ds(pn, _LTOK)], sem)

        G = 12
        def grp_body(g, c3):
            cb = g * (16 * G)
            def tok_body(l2, accs):
                l = p + 2 * l2
                a = tuple(
                    accs[j] + buf_v[l, pl.ds(cb + 16 * j, 16)]
                    for j in range(G))
                return tuple(
                    a[j] + buf_v[l + 1, pl.ds(cb + 16 * j, 16)]
                    for j in range(G))
            z = jnp.zeros((16,), jnp.float32)
            accs = lax.fori_loop(0, _LTOK // 2, tok_body, (z,) * G)
            for j in range(G):
                outv_v[i, pl.ds(cb + 16 * j, 16)] = accs[j]
            return c3
        lax.fori_loop(0, _NCH // G, grp_body, 0)
        return carry

    lax.fori_loop(0, _RPW, sent_body, 0)
    pltpu.sync_copy(outv_v, out_hbm.at[pl.ds(base, _RPW)])


def _pool_sc(sentences_hidden3):
    mesh = plsc.VectorSubcoreMesh(core_axis_name="c", subcore_axis_name="s")
    kfn = pl.kernel(
        _pool_sc_body,
        mesh=mesh,
        out_type=jax.ShapeDtypeStruct((_R_SC, _DH), jnp.float32),
        scratch_types=[
            pltpu.VMEM((2 * _LTOK, _DH), jnp.float32),
            pltpu.VMEM((_RPW, _DH), jnp.float32),
            pltpu.SemaphoreType.DMA,
        ],
    )
    return kfn(sentences_hidden3)


def _pool_body(s_ref, out_ref):
    out_ref[...] = jnp.sum(s_ref[...], axis=1)


def _pool_tc_tail(sentences_hidden3, rows_per_block=16):
    BS, L, DH = sentences_hidden3.shape
    ntail = BS - _R_SC
    nblk = ntail // rows_per_block
    off = _R_SC // rows_per_block
    return pl.pallas_call(
        _pool_body,
        grid=(nblk,),
        in_specs=[
            pl.BlockSpec((rows_per_block, L, DH), lambda i: (i + off, 0, 0)),
        ],
        out_specs=pl.BlockSpec((rows_per_block, DH), lambda i: (i, 0)),
        out_shape=jax.ShapeDtypeStruct((ntail, DH), jnp.float32),
    )(sentences_hidden3)


def _fused_body(ps_sc_ref, ps_tc_ref, mask_ref, ht_ref, nq_ref, adj_ref,
                et_ref, W_hp_ref, b_hp_ref, W_ql_ref, b_ql_ref, W_kl_ref,
                b_kl_ref, g_q_ref, beta_q_ref, g_k_ref, beta_k_ref,
                flag_ref, edge_ref, Wq_ref, Wk_ref, Wv_ref, We_ref,
                hidden_ref, recall_ref, *, B, S):
    ps = jnp.concatenate([ps_sc_ref[...], ps_tc_ref[...]], axis=0)
    msum = jnp.sum(mask_ref[...], axis=1, keepdims=True)
    inv = 1.0 / jnp.where(msum != 0.0, msum, 1.0)
    node = jnp.dot(ps, W_hp_ref[...],
                   preferred_element_type=jnp.float32) * inv + b_hp_ref[...]
    ht = ht_ref[...].astype(jnp.float32)
    f0 = flag_ref[0:1, :]
    f1 = flag_ref[1:2, :]
    h = node + f0 + ht * (f1 - f0)
    q = jnp.dot(h, Wq_ref[...], preferred_element_type=jnp.float32)
    k = jnp.dot(h, Wk_ref[...], preferred_element_type=jnp.float32)
    v = jnp.dot(h, Wv_ref[...], preferred_element_type=jnp.float32)
    ew = jnp.dot(edge_ref[...], We_ref[...],
                 preferred_element_type=jnp.float32)
    ql = jnp.dot(nq_ref[...], W_ql_ref[...],
                 preferred_element_type=jnp.float32) + b_ql_ref[...]
    mu = jnp.mean(ql, axis=-1, keepdims=True)
    var = jnp.mean((ql - mu) ** 2, axis=-1, keepdims=True)
    query = ((ql - mu) / jnp.sqrt(var + 1e-5)) * g_q_ref[...] \
        + beta_q_ref[...]

    dn = (((1,), (1,)), ((), ()))
    qe = lax.dot_general(q, ew, dn,
                         preferred_element_type=jnp.float32)
    isq = 1.0 / math.sqrt(float(D))
    neg = jnp.float32(-1e9)
    outs = []
    for j in range(B):
        sl = slice(j * S, (j + 1) * S)
        adj = adj_ref[j]
        et = et_ref[j]
        scores = lax.dot_general(q[sl], k[sl], dn,
                                 preferred_element_type=jnp.float32)
        esc = jnp.zeros_like(scores)
        qej = qe[sl]
        for t in range(5):
            esc = jnp.where(et == t,
                            jnp.broadcast_to(qej[:, t:t + 1], scores.shape),
                            esc)
        scores = (scores + esc) * isq
        scores = jnp.where(adj > 0, scores, neg)
        mx = jnp.max(scores, axis=-1, keepdims=True)
        p = jnp.exp(scores - mx)
        attn = p / jnp.sum(p, axis=-1, keepdims=True)
        row_has = (jnp.sum(adj.astype(jnp.float32), axis=-1, keepdims=True)
                   > 0.0).astype(jnp.float32)
        attn = attn * row_has
        outs.append(jnp.dot(attn, v[sl],
                            preferred_element_type=jnp.float32))
    hidden = jnp.concatenate(outs, axis=0) + h
    for j in range(B):
        hidden_ref[j] = hidden[j * S:(j + 1) * S]
    kl = jnp.dot(hidden, W_kl_ref[...],
                 preferred_element_type=jnp.float32) + b_kl_ref[...]
    mu = jnp.mean(kl, axis=-1, keepdims=True)
    var = jnp.mean((kl - mu) ** 2, axis=-1, keepdims=True)
    key = ((kl - mu) / jnp.sqrt(var + 1e-5)) * g_k_ref[...] + beta_k_ref[...]
    pad = (jnp.sum(mask_ref[...], axis=-1) != 0.0).astype(jnp.float32)
    for j in range(B):
        sl = slice(j * S, (j + 1) * S)
        logits = jnp.sum(key[sl] * query[j:j + 1, :], axis=-1)
        recall_ref[j:j + 1, :] = (jax.nn.sigmoid(logits) * pad[sl])[None, :]


def _fused(ps_sc, ps_tc, mask, head_flat, node_query, adj, et,
           W_hp, b_hp, W_ql, b_ql, W_kl, b_kl, g_q, beta_q, g_k, beta_k,
           flag_embed, edge_embed, Wq, Wk, Wv, We):
    B = adj.shape[0]
    S = adj.shape[1]
    outs = (
        jax.ShapeDtypeStruct((B, S, D), jnp.float32),
        jax.ShapeDtypeStruct((B, S), jnp.float32),
    )
    return pl.pallas_call(
        functools.partial(_fused_body, B=B, S=S), out_shape=outs)(
        ps_sc, ps_tc, mask, head_flat, node_query, adj, et, W_hp, b_hp,
        W_ql, b_ql, W_kl, b_kl, g_q, beta_q, g_k, beta_k, flag_embed,
        edge_embed, Wq, Wk, Wv, We)


def kernel(sentences_hidden, sentences_num, sentences_mask,
           sent_adjacent_matrix, head_type, edge_type, node_query,
           W_hp, b_hp, W_ql, b_ql, W_kl, b_kl, g_q, beta_q, g_k, beta_k,
           flag_embed, edge_embed, Wq, Wk, Wv, We):
    BS, L, DH = sentences_hidden.shape
    B = sentences_num.shape[0]
    S = BS // B

    s3 = sentences_hidden.reshape(BS, L, DH)
    ps_sc = _pool_sc(s3)
    ps_tc = _pool_tc_tail(s3)

    head_flat = head_type.reshape(BS, 1).astype(jnp.int32)
    r1 = lambda x: x.reshape(1, -1)
    adj = sent_adjacent_matrix.astype(jnp.int32)
    et = edge_type.astype(jnp.int32)
    hidden, recall = _fused(
        ps_sc, ps_tc, sentences_mask, head_flat, node_query, adj, et,
        W_hp, r1(b_hp), W_ql, r1(b_ql), W_kl, r1(b_kl), r1(g_q), r1(beta_q),
        r1(g_k), r1(beta_k), flag_embed, edge_embed, Wq, Wk, Wv, We)
    return recall, hidden


# speedup vs baseline: 1.0102x; 1.0047x over previous
import functools
import math

import jax
import jax.numpy as jnp
from jax import lax
from jax.experimental import pallas as pl
from jax.experimental.pallas import tpu as pltpu
from jax.experimental.pallas import tpu_sc as plsc

D = 512

_NW = 32
_R_SC = 512
_RPW = _R_SC // _NW
_LTOK = 64
_DH = 768
_NCH = _DH // 16


def _pool_sc_body(s_hbm, out_hbm, buf_v, outv_v, sem):
    wid = lax.axis_index("s") * 2 + lax.axis_index("c")
    base = wid * _RPW

    pltpu.async_copy(s_hbm.at[base], buf_v.at[pl.ds(0, _LTOK)], sem)

    def sent_body(i, carry):
        p = lax.rem(i, 2) * _LTOK
        row = base + i
        pltpu.make_async_copy(
            s_hbm.at[row], buf_v.at[pl.ds(p, _LTOK)], sem).wait()

        @pl.when(i + 1 < _RPW)
        def _():
            pn = lax.rem(i + 1, 2) * _LTOK
            pltpu.async_copy(s_hbm.at[row + 1],
                             buf_v.at[pl.ds(pn, _LTOK)], sem)

        G = 12
        def grp_body(g, c3):
            cb = g * (16 * G)
            def tok_body(l2, accs):
                l = p + 2 * l2
                a = tuple(
                    accs[j] + buf_v[l, pl.ds(cb + 16 * j, 16)]
                    for j in range(G))
                return tuple(
                    a[j] + buf_v[l + 1, pl.ds(cb + 16 * j, 16)]
                    for j in range(G))
            z = jnp.zeros((16,), jnp.float32)
            accs = lax.fori_loop(0, _LTOK // 2, tok_body, (z,) * G)
            for j in range(G):
                outv_v[i, pl.ds(cb + 16 * j, 16)] = accs[j]
            return c3
        lax.fori_loop(0, _NCH // G, grp_body, 0)
        return carry

    lax.fori_loop(0, _RPW, sent_body, 0)
    pltpu.sync_copy(outv_v, out_hbm.at[pl.ds(base, _RPW)])


def _pool_sc(sentences_hidden3):
    mesh = plsc.VectorSubcoreMesh(core_axis_name="c", subcore_axis_name="s")
    kfn = pl.kernel(
        _pool_sc_body,
        mesh=mesh,
        out_type=jax.ShapeDtypeStruct((_R_SC, _DH), jnp.float32),
        scratch_types=[
            pltpu.VMEM((2 * _LTOK, _DH), jnp.float32),
            pltpu.VMEM((_RPW, _DH), jnp.float32),
            pltpu.SemaphoreType.DMA,
        ],
    )
    return kfn(sentences_hidden3)


def _pool_body(s_ref, out_ref):
    out_ref[...] = jnp.sum(s_ref[...], axis=1)


def _pool_tc_tail(sentences_hidden3, rows_per_block=32):
    BS, L, DH = sentences_hidden3.shape
    ntail = BS - _R_SC
    nblk = ntail // rows_per_block
    off = _R_SC // rows_per_block
    return pl.pallas_call(
        _pool_body,
        grid=(nblk,),
        in_specs=[
            pl.BlockSpec((rows_per_block, L, DH), lambda i: (i + off, 0, 0)),
        ],
        out_specs=pl.BlockSpec((rows_per_block, DH), lambda i: (i, 0)),
        out_shape=jax.ShapeDtypeStruct((ntail, DH), jnp.float32),
    )(sentences_hidden3)


def _fused_body(ps_sc_ref, ps_tc_ref, mask_ref, ht_ref, nq_ref, adj_ref,
                et_ref, W_hp_ref, b_hp_ref, W_ql_ref, b_ql_ref, W_kl_ref,
                b_kl_ref, g_q_ref, beta_q_ref, g_k_ref, beta_k_ref,
                flag_ref, edge_ref, Wq_ref, Wk_ref, Wv_ref, We_ref,
                hidden_ref, recall_ref, *, B, S):
    ps = jnp.concatenate([ps_sc_ref[...], ps_tc_ref[...]], axis=0)
    msum = jnp.sum(mask_ref[...], axis=1, keepdims=True)
    inv = 1.0 / jnp.where(msum != 0.0, msum, 1.0)
    node = jnp.dot(ps, W_hp_ref[...],
                   preferred_element_type=jnp.float32) * inv + b_hp_ref[...]
    ht = ht_ref[...].astype(jnp.float32)
    f0 = flag_ref[0:1, :]
    f1 = flag_ref[1:2, :]
    h = node + f0 + ht * (f1 - f0)
    q = jnp.dot(h, Wq_ref[...], preferred_element_type=jnp.float32)
    k = jnp.dot(h, Wk_ref[...], preferred_element_type=jnp.float32)
    v = jnp.dot(h, Wv_ref[...], preferred_element_type=jnp.float32)
    ew = jnp.dot(edge_ref[...], We_ref[...],
                 preferred_element_type=jnp.float32)
    ql = jnp.dot(nq_ref[...], W_ql_ref[...],
                 preferred_element_type=jnp.float32) + b_ql_ref[...]
    mu = jnp.mean(ql, axis=-1, keepdims=True)
    var = jnp.mean((ql - mu) ** 2, axis=-1, keepdims=True)
    query = ((ql - mu) / jnp.sqrt(var + 1e-5)) * g_q_ref[...] \
        + beta_q_ref[...]

    dn = (((1,), (1,)), ((), ()))
    qe = lax.dot_general(q, ew, dn,
                         preferred_element_type=jnp.float32)
    isq = 1.0 / math.sqrt(float(D))
    neg = jnp.float32(-1e9)
    outs = []
    for j in range(B):
        sl = slice(j * S, (j + 1) * S)
        adj = adj_ref[j]
        et = et_ref[j]
        scores = lax.dot_general(q[sl], k[sl], dn,
                                 preferred_element_type=jnp.float32)
        esc = jnp.zeros_like(scores)
        qej = qe[sl]
        for t in range(5):
            esc = jnp.where(et == t,
                            jnp.broadcast_to(qej[:, t:t + 1], scores.shape),
                            esc)
        scores = (scores + esc) * isq
        scores = jnp.where(adj > 0, scores, neg)
        mx = jnp.max(scores, axis=-1, keepdims=True)
        p = jnp.exp(scores - mx)
        attn = p / jnp.sum(p, axis=-1, keepdims=True)
        row_has = (jnp.sum(adj.astype(jnp.float32), axis=-1, keepdims=True)
                   > 0.0).astype(jnp.float32)
        attn = attn * row_has
        outs.append(jnp.dot(attn, v[sl],
                            preferred_element_type=jnp.float32))
    hidden = jnp.concatenate(outs, axis=0) + h
    for j in range(B):
        hidden_ref[j] = hidden[j * S:(j + 1) * S]
    kl = jnp.dot(hidden, W_kl_ref[...],
                 preferred_element_type=jnp.float32) + b_kl_ref[...]
    mu = jnp.mean(kl, axis=-1, keepdims=True)
    var = jnp.mean((kl - mu) ** 2, axis=-1, keepdims=True)
    key = ((kl - mu) / jnp.sqrt(var + 1e-5)) * g_k_ref[...] + beta_k_ref[...]
    pad = (jnp.sum(mask_ref[...], axis=-1) != 0.0).astype(jnp.float32)
    for j in range(B):
        sl = slice(j * S, (j + 1) * S)
        logits = jnp.sum(key[sl] * query[j:j + 1, :], axis=-1)
        recall_ref[j:j + 1, :] = (jax.nn.sigmoid(logits) * pad[sl])[None, :]


def _fused(ps_sc, ps_tc, mask, head_flat, node_query, adj, et,
           W_hp, b_hp, W_ql, b_ql, W_kl, b_kl, g_q, beta_q, g_k, beta_k,
           flag_embed, edge_embed, Wq, Wk, Wv, We):
    B = adj.shape[0]
    S = adj.shape[1]
    outs = (
        jax.ShapeDtypeStruct((B, S, D), jnp.float32),
        jax.ShapeDtypeStruct((B, S), jnp.float32),
    )
    return pl.pallas_call(
        functools.partial(_fused_body, B=B, S=S), out_shape=outs)(
        ps_sc, ps_tc, mask, head_flat, node_query, adj, et, W_hp, b_hp,
        W_ql, b_ql, W_kl, b_kl, g_q, beta_q, g_k, beta_k, flag_embed,
        edge_embed, Wq, Wk, Wv, We)


def kernel(sentences_hidden, sentences_num, sentences_mask,
           sent_adjacent_matrix, head_type, edge_type, node_query,
           W_hp, b_hp, W_ql, b_ql, W_kl, b_kl, g_q, beta_q, g_k, beta_k,
           flag_embed, edge_embed, Wq, Wk, Wv, We):
    BS, L, DH = sentences_hidden.shape
    B = sentences_num.shape[0]
    S = BS // B

    s3 = sentences_hidden.reshape(BS, L, DH)
    ps_sc = _pool_sc(s3)
    ps_tc = _pool_tc_tail(s3)

    head_flat = head_type.reshape(BS, 1).astype(jnp.int32)
    r1 = lambda x: x.reshape(1, -1)
    adj = sent_adjacent_matrix.astype(jnp.int32)
    et = edge_type.astype(jnp.int32)
    hidden, recall = _fused(
        ps_sc, ps_tc, sentences_mask, head_flat, node_query, adj, et,
        W_hp, r1(b_hp), W_ql, r1(b_ql), W_kl, r1(b_kl), r1(g_q), r1(beta_q),
        r1(g_k), r1(beta_k), flag_embed, edge_embed, Wq, Wk, Wv, We)
    return recall, hidden
